# Initial kernel scaffold; baseline (speedup 1.0000x reference)
import functools

import jax
import jax.numpy as jnp
from jax import lax
from jax.experimental import pallas as pl
from jax.experimental.pallas import tpu as pltpu
from jax.experimental.pallas import tpu_sc as plsc

N = 10000
D = 128
F = 128
E = 320000
NEG_SLOPE = 0.2

NC = 2
NS = 16
NW = NC * NS
L = 16

NPAD = 10240
CHUNK = 128
NCHUNKS = E // CHUNK
KMAX = -(-NCHUNKS // NW)
ROWS_PER_TILE = NPAD // NS
DEN_ROWS = NPAD // 128


def _mm_body(x_ref, w_ref, a_ref, feat_ref, el2_ref):
    xv = x_ref[...]
    wv = w_ref[...]
    feat_ref[...] = lax.dot_general(
        xv, wv, (((1,), (1,)), ((), ())), preferred_element_type=jnp.float32)
    aw = lax.dot_general(
        a_ref[...], wv, (((1,), (0,)), ((), ())),
        preferred_element_type=jnp.float32)
    el2_ref[...] = lax.dot_general(
        aw, xv, (((1,), (1,)), ((), ())), preferred_element_type=jnp.float32)


def _matmul(x_pad, W, a8):
    return pl.pallas_call(
        _mm_body,
        out_shape=[
            jax.ShapeDtypeStruct((NPAD, F), jnp.float32),
            jax.ShapeDtypeStruct((8, NPAD), jnp.float32),
        ],
    )(x_pad, W, a8)


def _edge_body(src_hbm, dst_hbm, el2_hbm, feat_hbm, outp_hbm, denp_hbm,
               el_v, er_v, den2_v, srcidx_v, dstidx_v, w_v, rows_v, iden_v,
               out_s, den_s, sem):
    cid = lax.axis_index("c")
    sid = lax.axis_index("s")
    wid = cid * NS + sid

    zeros16 = jnp.zeros((L,), jnp.float32)

    def _zrow(i, c):
        for j in range(F // L):
            rows_v[i, pl.ds(j * L, L)] = zeros16
        return c
    lax.fori_loop(0, CHUNK, _zrow, 0)

    def _zden(i, c):
        for j in range(128 // L):
            den2_v[i, pl.ds(j * L, L)] = zeros16
        return c
    lax.fori_loop(0, DEN_ROWS, _zden, 0)

    r0 = sid * ROWS_PER_TILE
    for k in range(ROWS_PER_TILE // CHUNK):
        pltpu.sync_copy(rows_v, out_s.at[pl.ds(r0 + k * CHUNK, CHUNK)])

    @pl.when(sid == 0)
    def _():
        pltpu.sync_copy(den2_v, den_s)

    for j in range(DEN_ROWS // L):
        iden_v[pl.ds(j * L, L)] = lax.iota(jnp.int32, L) + j * L

    pltpu.sync_copy(el2_hbm.at[0], el_v)
    pltpu.sync_copy(el2_hbm.at[1], er_v)

    plsc.subcore_barrier()

    def _chunk(k, c):
        ch = k * NW + wid

        @pl.when(ch < NCHUNKS)
        def _():
            off = ch * CHUNK
            pltpu.sync_copy(src_hbm.at[pl.ds(off, CHUNK)], srcidx_v)
            pltpu.sync_copy(dst_hbm.at[pl.ds(off, CHUNK)], dstidx_v)
            gather = pltpu.async_copy(feat_hbm.at[srcidx_v], rows_v, sem)
            for j in range(CHUNK // L):
                sidx = srcidx_v[pl.ds(j * L, L)]
                didx = dstidx_v[pl.ds(j * L, L)]
                e = plsc.load_gather(el_v, [sidx]) + plsc.load_gather(er_v, [didx])
                e = jnp.where(e >= 0, e, NEG_SLOPE * e)
                w = jnp.exp(e)
                w_v[pl.ds(j * L, L)] = w
                plsc.addupdate_scatter(
                    den2_v, [lax.shift_right_logical(didx, 7), didx & 127], w)
            gather.wait()

            def _scale(i, c2):
                wv = w_v[i]
                for j in range(F // L):
                    rows_v[i, pl.ds(j * L, L)] = rows_v[i, pl.ds(j * L, L)] * wv
                return c2
            lax.fori_loop(0, CHUNK, _scale, 0)

            pltpu.sync_copy(rows_v, out_s.at[dstidx_v], add=True)
        return c

    lax.fori_loop(0, KMAX, _chunk, 0)

    plsc.subcore_barrier()

    pltpu.sync_copy(den2_v, den_s.at[iden_v], add=True)

    plsc.subcore_barrier()

    pltpu.sync_copy(out_s.at[pl.ds(r0, ROWS_PER_TILE)],
                    outp_hbm.at[cid, pl.ds(r0, ROWS_PER_TILE)])

    @pl.when(sid == 0)
    def _():
        pltpu.sync_copy(den_s, denp_hbm.at[cid])


def _edge_phase(src, dst, el2, feat):
    return pl.kernel(
        _edge_body,
        out_type=[
            jax.ShapeDtypeStruct((NC, NPAD, F), jnp.float32),
            jax.ShapeDtypeStruct((NC, DEN_ROWS, 128), jnp.float32),
        ],
        mesh=plsc.VectorSubcoreMesh(core_axis_name="c", subcore_axis_name="s"),
        scratch_types=[
            pltpu.VMEM((NPAD,), jnp.float32),
            pltpu.VMEM((NPAD,), jnp.float32),
            pltpu.VMEM((DEN_ROWS, 128), jnp.float32),
            pltpu.VMEM((CHUNK,), jnp.int32),
            pltpu.VMEM((CHUNK,), jnp.int32),
            pltpu.VMEM((CHUNK,), jnp.float32),
            pltpu.VMEM((CHUNK, F), jnp.float32),
            pltpu.VMEM((DEN_ROWS,), jnp.int32),
            pltpu.VMEM_SHARED((NPAD, F), jnp.float32),
            pltpu.VMEM_SHARED((DEN_ROWS, 128), jnp.float32),
            pltpu.SemaphoreType.DMA,
        ],
    )(src, dst, el2, feat)


def _fin_body(op_ref, dp_ref, b_ref, o_ref):
    s = op_ref[0] + op_ref[1]
    dsum = dp_ref[0] + dp_ref[1]
    rec = 1.0 / (dsum + 1e-16)
    o_ref[...] = s[:N] * rec[:N] + b_ref[...]


def _finalize(outp, denp, bias2d):
    return pl.pallas_call(
        _fin_body,
        out_shape=jax.ShapeDtypeStruct((N, F), jnp.float32),
    )(outp, denp, bias2d)


def kernel(x, edge_index, W, attn_l, attn_r, bias):
    src = edge_index[0].astype(jnp.int32)
    dst = edge_index[1].astype(jnp.int32)
    x_pad = jnp.pad(x, ((0, NPAD - N), (0, 0)))
    a8 = jnp.zeros((8, F), jnp.float32)
    a8 = a8.at[0].set(attn_l.reshape(F)).at[1].set(attn_r.reshape(F))

    feat, el2 = _matmul(x_pad, W, a8)
    outp, denp = _edge_phase(src, dst, el2, feat)
    denp_r = denp.reshape(NC, NPAD, 1)
    out = _finalize(outp, denp_r, bias.reshape(1, F))
    return out


# speedup vs baseline: 51.7253x; 51.7253x over previous
import functools

import jax
import jax.numpy as jnp
from jax import lax
from jax.experimental import pallas as pl
from jax.experimental.pallas import tpu as pltpu
from jax.experimental.pallas import tpu_sc as plsc

N = 10000
D = 128
F = 128
E = 320000
NEG_SLOPE = 0.2

NC = 2
NS = 16
NW = NC * NS
L = 16

NPAD = 10240
CHUNK = 80
KM = E // NW // CHUNK
SBLK = 8
BLK_E = SBLK * CHUNK
NBLK = -(-KM // SBLK)
NCH_PAD = NW * SBLK * NBLK
E_PAD = NCH_PAD * CHUNK
ROWS_PER_TILE = NPAD // NS


def _mm_body(x_ref, w_ref, a_ref, feat_ref, el2_ref):
    xv = x_ref[...]
    wv = w_ref[...]
    feat_ref[...] = lax.dot_general(
        xv, wv, (((1,), (1,)), ((), ())), preferred_element_type=jnp.float32)
    aw = lax.dot_general(
        a_ref[...], wv, (((1,), (0,)), ((), ())),
        preferred_element_type=jnp.float32)
    el2_ref[...] = lax.dot_general(
        aw, xv, (((1,), (1,)), ((), ())), preferred_element_type=jnp.float32)


def _matmul(x_pad, W, a8):
    return pl.pallas_call(
        _mm_body,
        out_shape=[
            jax.ShapeDtypeStruct((NPAD, F), jnp.float32),
            jax.ShapeDtypeStruct((8, NPAD), jnp.float32),
        ],
    )(x_pad, W, a8)


def _edge_body(src_hbm, dst_hbm, el2_hbm, feat_hbm, outp_hbm, denp_hbm,
               el_v, er_v, srcb_v, dstb_v, w_v, rows_v,
               out_s, den_s, gsem, ssem, dsem, bsem):
    cid = lax.axis_index("c")
    sid = lax.axis_index("s")
    wid = cid * NS + sid
    c0 = wid * (SBLK * NBLK)

    zeros16 = jnp.zeros((L,), jnp.float32)

    def _zrow(i, c):
        for j in range(F // L):
            rows_v[0, i, pl.ds(j * L, L)] = zeros16
        return c
    lax.fori_loop(0, CHUNK, _zrow, 0)

    for j in range(CHUNK // L):
        w_v[0, pl.ds(j * L, L)] = zeros16

    r0 = sid * ROWS_PER_TILE
    for k in range(ROWS_PER_TILE // CHUNK):
        pltpu.sync_copy(rows_v.at[0], out_s.at[pl.ds(r0 + k * CHUNK, CHUNK)])
        pltpu.sync_copy(w_v.at[0], den_s.at[pl.ds(r0 + k * CHUNK, CHUNK)])

    pltpu.sync_copy(el2_hbm.at[0], el_v)
    pltpu.sync_copy(el2_hbm.at[1], er_v)

    plsc.subcore_barrier()

    def _stage(blk, bb, sync):
        copy = pltpu.sync_copy if sync else (
            lambda s, d: pltpu.async_copy(s, d, bsem.at[bb]))
        copy(src_hbm.at[pl.ds(c0 + blk * SBLK, SBLK)], srcb_v.at[bb])
        copy(dst_hbm.at[pl.ds(c0 + blk * SBLK, SBLK)], dstb_v.at[bb])

    def _fire_gather(k, b, bb, pos):
        pltpu.async_copy(
            feat_hbm.at[srcb_v.at[bb, pos]], rows_v.at[b], gsem.at[b])

    _stage(0, 0, True)
    _fire_gather(0, 0, 0, 0)

    def _chunk(k, c):
        b = lax.rem(k, 2)
        blk = lax.div(k, SBLK)
        pos = lax.rem(k, SBLK)
        bb = lax.rem(blk, 2)
        bn = 1 - b

        @pl.when(jnp.logical_and(k >= 1, k + 1 < KM))
        def _():
            pltpu.make_async_copy(
                rows_v.at[bn], out_s.at[dstb_v.at[bb, pos]],
                ssem.at[bn]).wait()
            pltpu.make_async_copy(
                w_v.at[bn], den_s.at[dstb_v.at[bb, pos]],
                dsem.at[bn]).wait()

        @pl.when(jnp.logical_and(pos == 0, blk + 1 < NBLK))
        def _():
            _stage(blk + 1, 1 - bb, False)

        @pl.when(k + 1 < KM)
        def _():
            @pl.when(pos == SBLK - 1)
            def _():
                pltpu.make_async_copy(
                    src_hbm.at[pl.ds(0, SBLK)], srcb_v.at[1 - bb],
                    bsem.at[1 - bb]).wait()
                pltpu.make_async_copy(
                    dst_hbm.at[pl.ds(0, SBLK)], dstb_v.at[1 - bb],
                    bsem.at[1 - bb]).wait()

            kn = k + 1
            _fire_gather(kn, bn, lax.rem(lax.div(kn, SBLK), 2),
                         lax.rem(kn, SBLK))

        pltpu.make_async_copy(
            feat_hbm.at[srcb_v.at[bb, pos]], rows_v.at[b], gsem.at[b]).wait()

        for j in range(CHUNK // L):
            sidx = srcb_v[bb, pos, pl.ds(j * L, L)]
            didx = dstb_v[bb, pos, pl.ds(j * L, L)]
            e = plsc.load_gather(el_v, [sidx]) + plsc.load_gather(er_v, [didx])
            e = jnp.where(e >= 0, e, NEG_SLOPE * e)
            w = jnp.exp(e)
            w_v[b, pl.ds(j * L, L)] = w
            for r in range(L):
                i = j * L + r
                wr = jnp.broadcast_to(w[r], (L,))
                for f0 in range(F // L):
                    rows_v[b, i, pl.ds(f0 * L, L)] = (
                        rows_v[b, i, pl.ds(f0 * L, L)] * wr)

        pltpu.async_copy(
            rows_v.at[b], out_s.at[dstb_v.at[bb, pos]], ssem.at[b], add=True)
        pltpu.async_copy(
            w_v.at[b], den_s.at[dstb_v.at[bb, pos]], dsem.at[b], add=True)
        return c

    lax.fori_loop(0, KM, _chunk, 0)

    for t in (KM - 2, KM - 1):
        bt, bbt, post = t % 2, (t // SBLK) % 2, t % SBLK
        pltpu.make_async_copy(
            rows_v.at[bt], out_s.at[dstb_v.at[bbt, post]], ssem.at[bt]).wait()
        pltpu.make_async_copy(
            w_v.at[bt], den_s.at[dstb_v.at[bbt, post]], dsem.at[bt]).wait()

    plsc.subcore_barrier()

    pltpu.sync_copy(out_s.at[pl.ds(r0, ROWS_PER_TILE)],
                    outp_hbm.at[cid, pl.ds(r0, ROWS_PER_TILE)])
    pltpu.sync_copy(den_s.at[pl.ds(r0, ROWS_PER_TILE)],
                    denp_hbm.at[cid, pl.ds(r0, ROWS_PER_TILE)])


def _edge_phase(src2, dst2, el2, feat):
    return pl.kernel(
        _edge_body,
        out_type=[
            jax.ShapeDtypeStruct((NC, NPAD, F), jnp.float32),
            jax.ShapeDtypeStruct((NC, NPAD), jnp.float32),
        ],
        mesh=plsc.VectorSubcoreMesh(core_axis_name="c", subcore_axis_name="s"),
        compiler_params=pltpu.CompilerParams(needs_layout_passes=False),
        scratch_types=[
            pltpu.VMEM((NPAD,), jnp.float32),
            pltpu.VMEM((NPAD,), jnp.float32),
            pltpu.VMEM((2, SBLK, CHUNK), jnp.int32),
            pltpu.VMEM((2, SBLK, CHUNK), jnp.int32),
            pltpu.VMEM((2, CHUNK), jnp.float32),
            pltpu.VMEM((2, CHUNK, F), jnp.float32),
            pltpu.VMEM_SHARED((NPAD, F), jnp.float32),
            pltpu.VMEM_SHARED((NPAD,), jnp.float32),
            pltpu.SemaphoreType.DMA((2,)),
            pltpu.SemaphoreType.DMA((2,)),
            pltpu.SemaphoreType.DMA((2,)),
            pltpu.SemaphoreType.DMA((2,)),
        ],
    )(src2, dst2, el2, feat)


def _fin_body(op_ref, dp_ref, b_ref, o_ref):
    s = op_ref[0] + op_ref[1]
    dsum = dp_ref[0] + dp_ref[1]
    rec = 1.0 / (dsum + 1e-16)
    o_ref[...] = s[:N] * rec[:N] + b_ref[...]


def _finalize(outp, denp, bias2d):
    return pl.pallas_call(
        _fin_body,
        out_shape=jax.ShapeDtypeStruct((N, F), jnp.float32),
    )(outp, denp, bias2d)


def kernel(x, edge_index, W, attn_l, attn_r, bias):
    epw = E // NW
    epw_pad = SBLK * NBLK * CHUNK
    src = edge_index[0].astype(jnp.int32)
    dst = edge_index[1].astype(jnp.int32)
    src2 = jnp.pad(src.reshape(NW, epw), ((0, 0), (0, epw_pad - epw)))
    src2 = src2.reshape(NCH_PAD, CHUNK)
    dst2 = jnp.pad(dst.reshape(NW, epw), ((0, 0), (0, epw_pad - epw)))
    dst2 = dst2.reshape(NCH_PAD, CHUNK)
    x_pad = jnp.pad(x, ((0, NPAD - N), (0, 0)))
    a8 = jnp.zeros((8, F), jnp.float32)
    a8 = a8.at[0].set(attn_l.reshape(F)).at[1].set(attn_r.reshape(F))

    feat, el2 = _matmul(x_pad, W, a8)
    outp, denp = _edge_phase(src2, dst2, el2, feat)
    denp_r = denp.reshape(NC, NPAD, 1)
    out = _finalize(outp, denp_r, bias.reshape(1, F))
    return out
